# baseline (speedup 1.0000x reference)
import functools

import jax
import jax.numpy as jnp
from jax import lax
from jax.experimental import pallas as pl
from jax.experimental.pallas import tpu as pltpu
from jax.experimental.pallas import tpu_sc as plsc

B = 16384
D = 32
L = 16
NC = 2
NS = 16
NW = NC * NS
BPW = B // NW
CH = 128
NCH = BPW // CH

_mesh = plsc.VectorSubcoreMesh(core_axis_name="c", subcore_axis_name="s")


@functools.partial(
    pl.kernel,
    mesh=_mesh,
    out_type=jax.ShapeDtypeStruct((B,), jnp.float32),
    compiler_params=pltpu.CompilerParams(
        needs_layout_passes=False, use_tc_tiling_on_sc=False),
    scratch_types=[
        pltpu.VMEM((NCH, CH), jnp.int32),
        pltpu.VMEM((NCH, CH), jnp.int32),
        pltpu.VMEM((BPW, D), jnp.float32),
        pltpu.VMEM((BPW, D), jnp.float32),
        pltpu.VMEM((BPW,), jnp.float32),
        pltpu.SemaphoreType.DMA,
    ],
)
def _sc_dot(users_hbm, items_hbm, uf_hbm, if_hbm, out_hbm,
            uidx_v, iidx_v, urows_v, irows_v, out_v, sem):
    wid = lax.axis_index("s") * NC + lax.axis_index("c")

    pltpu.sync_copy(users_hbm.at[wid], uidx_v)
    pltpu.sync_copy(items_hbm.at[wid], iidx_v)

    copies = []
    for j in range(NCH):
        copies.append(
            pltpu.async_copy(uf_hbm.at[uidx_v.at[j]],
                             urows_v.at[pl.ds(j * CH, CH)], sem))
        copies.append(
            pltpu.async_copy(if_hbm.at[iidx_v.at[j]],
                             irows_v.at[pl.ds(j * CH, CH)], sem))
    for c in copies:
        c.wait()

    lanes = lax.iota(jnp.int32, L)

    def body(g, carry):
        base = g * L
        acc = jnp.zeros((L,), jnp.float32)
        for i in range(L):
            r = base + i
            u0 = urows_v[r, pl.ds(0, L)]
            u1 = urows_v[r, pl.ds(L, L)]
            v0 = irows_v[r, pl.ds(0, L)]
            v1 = irows_v[r, pl.ds(L, L)]
            tot = jnp.sum(u0 * v0 + u1 * v1)
            acc = jnp.where(lanes == i, tot, acc)
        out_v[pl.ds(base, L)] = acc
        return carry

    lax.fori_loop(0, BPW // L, body, 0)

    pltpu.sync_copy(out_v, out_hbm.at[pl.ds(wid * BPW, BPW)])


def kernel(data, user_factors, item_factors):
    users = data[:, 0].astype(jnp.int32).reshape(NW, NCH, CH)
    items = data[:, 1].astype(jnp.int32).reshape(NW, NCH, CH)
    return _sc_dot(users, items, user_factors, item_factors)


# speedup vs baseline: 10.8637x; 10.8637x over previous
import functools

import jax
import jax.numpy as jnp
from jax import lax
from jax.experimental import pallas as pl
from jax.experimental.pallas import tpu as pltpu
from jax.experimental.pallas import tpu_sc as plsc

B = 16384
D = 32
L = 16
NC = 2
NS = 16
NW = NC * NS
BPW = B // NW
N_ROWS = 1000000
TR = 4
WR = 8
NSLOT = 64
NB = BPW // NSLOT

_mesh = plsc.VectorSubcoreMesh(core_axis_name="c", subcore_axis_name="s")


@functools.partial(
    pl.kernel,
    mesh=_mesh,
    out_type=jax.ShapeDtypeStruct((B,), jnp.float32),
    compiler_params=pltpu.CompilerParams(
        needs_layout_passes=False, use_tc_tiling_on_sc=True),
    scratch_types=[
        pltpu.VMEM((BPW + L,), jnp.int32),
        pltpu.VMEM((BPW + L,), jnp.int32),
        pltpu.VMEM((TR, WR, L * NSLOT), jnp.float32),
        pltpu.VMEM((TR, WR, L * NSLOT), jnp.float32),
        pltpu.VMEM((BPW,), jnp.float32),
        pltpu.SemaphoreType.DMA,
    ],
)
def _sc_dot(users_hbm, items_hbm, uq_hbm, iq_hbm, out_hbm,
            uidx_v, iidx_v, u_scr, i_scr, out_v, sem):
    wid = lax.axis_index("s") * NC + lax.axis_index("c")

    pltpu.sync_copy(users_hbm.at[wid], uidx_v.at[pl.ds(0, BPW)])
    pltpu.sync_copy(items_hbm.at[wid], iidx_v.at[pl.ds(0, BPW)])

    lanes = lax.iota(jnp.int32, L)

    def batch_body(b, carry):
        def issue_body(s, carry2):
            iu = uidx_v[pl.ds(b * NSLOT + s, L)][0]
            ii = iidx_v[pl.ds(b * NSLOT + s, L)][0]
            src_u = pl.multiple_of(iu & -16, 128)
            src_i = pl.multiple_of(ii & -16, 128)
            dst_o = pl.multiple_of(s * L, 128)
            pltpu.make_async_copy(
                uq_hbm.at[:, :, pl.ds(src_u, L)],
                u_scr.at[:, :, pl.ds(dst_o, L)], sem).start()
            pltpu.make_async_copy(
                iq_hbm.at[:, :, pl.ds(src_i, L)],
                i_scr.at[:, :, pl.ds(dst_o, L)], sem).start()
            return carry2

        lax.fori_loop(0, NSLOT, issue_body, 0)

        def drain_body(s, carry2):
            pltpu.make_async_copy(
                uq_hbm.at[:, :, pl.ds(0, L)],
                u_scr.at[:, :, pl.ds(0, L)], sem).wait()
            pltpu.make_async_copy(
                iq_hbm.at[:, :, pl.ds(0, L)],
                i_scr.at[:, :, pl.ds(0, L)], sem).wait()
            return carry2

        lax.fori_loop(0, NSLOT, drain_body, 0)

        for g in range(NSLOT // L):
            uvec = uidx_v[pl.ds(b * NSLOT + g * L, L)]
            ivec = iidx_v[pl.ds(b * NSLOT + g * L, L)]
            lane_u = (g * L + lanes) * L + (uvec & (L - 1))
            lane_i = (g * L + lanes) * L + (ivec & (L - 1))
            acc = jnp.zeros((L,), jnp.float32)
            for tr in range(TR):
                trv = jnp.full((L,), tr, jnp.int32)
                for wr in range(WR):
                    wrv = jnp.full((L,), wr, jnp.int32)
                    u = plsc.load_gather(u_scr, [trv, wrv, lane_u])
                    v = plsc.load_gather(i_scr, [trv, wrv, lane_i])
                    acc = acc + u * v
            out_v[pl.ds(b * NSLOT + g * L, L)] = acc
        return carry

    lax.fori_loop(0, NB, batch_body, 0)

    pltpu.sync_copy(out_v, out_hbm.at[pl.ds(wid * BPW, BPW)])


def kernel(data, user_factors, item_factors):
    users = data[:, 0].astype(jnp.int32).reshape(NW, BPW)
    items = data[:, 1].astype(jnp.int32).reshape(NW, BPW)
    uq = user_factors.T.reshape(TR, WR, N_ROWS)
    iq = item_factors.T.reshape(TR, WR, N_ROWS)
    return _sc_dot(users, items, uq, iq)


# speedup vs baseline: 12.1711x; 1.1203x over previous
import functools

import jax
import jax.numpy as jnp
from jax import lax
from jax.experimental import pallas as pl
from jax.experimental.pallas import tpu as pltpu
from jax.experimental.pallas import tpu_sc as plsc

B = 16384
D = 32
L = 16
NC = 2
NS = 16
NW = NC * NS
BPW = B // NW
N_ROWS = 1000000
TR = 4
WR = 8
NSLOT = 64
FW = 8
NB = BPW // NSLOT

_mesh = plsc.VectorSubcoreMesh(core_axis_name="c", subcore_axis_name="s")


@functools.partial(
    pl.kernel,
    mesh=_mesh,
    out_type=jax.ShapeDtypeStruct((B,), jnp.float32),
    compiler_params=pltpu.CompilerParams(
        needs_layout_passes=False, use_tc_tiling_on_sc=True),
    scratch_types=[
        pltpu.VMEM((BPW + L,), jnp.int32),
        pltpu.VMEM((BPW + L,), jnp.int32),
        pltpu.VMEM((TR, WR, FW * NSLOT), jnp.float32),
        pltpu.VMEM((TR, WR, FW * NSLOT), jnp.float32),
        pltpu.VMEM((BPW,), jnp.float32),
        pltpu.SemaphoreType.DMA,
    ],
)
def _sc_dot(users_hbm, items_hbm, uq_hbm, iq_hbm, out_hbm,
            uidx_v, iidx_v, u_scr, i_scr, out_v, sem):
    wid = lax.axis_index("s") * NC + lax.axis_index("c")

    pltpu.sync_copy(users_hbm.at[wid], uidx_v.at[pl.ds(0, BPW)])
    pltpu.sync_copy(items_hbm.at[wid], iidx_v.at[pl.ds(0, BPW)])

    lanes = lax.iota(jnp.int32, L)

    def batch_body(b, carry):
        def issue_body(s, carry2):
            iu = uidx_v[pl.ds(b * NSLOT + s, L)][0]
            ii = iidx_v[pl.ds(b * NSLOT + s, L)][0]
            src_u = pl.multiple_of(iu & -FW, 128)
            src_i = pl.multiple_of(ii & -FW, 128)
            dst_o = pl.multiple_of(s * FW, 128)
            pltpu.make_async_copy(
                uq_hbm.at[:, :, pl.ds(src_u, FW)],
                u_scr.at[:, :, pl.ds(dst_o, FW)], sem).start()
            pltpu.make_async_copy(
                iq_hbm.at[:, :, pl.ds(src_i, FW)],
                i_scr.at[:, :, pl.ds(dst_o, FW)], sem).start()
            return carry2

        lax.fori_loop(0, NSLOT, issue_body, 0)

        def drain_body(s, carry2):
            pltpu.make_async_copy(
                uq_hbm.at[:, :, pl.ds(0, FW)],
                u_scr.at[:, :, pl.ds(0, FW)], sem).wait()
            pltpu.make_async_copy(
                iq_hbm.at[:, :, pl.ds(0, FW)],
                i_scr.at[:, :, pl.ds(0, FW)], sem).wait()
            return carry2

        lax.fori_loop(0, NSLOT, drain_body, 0)

        for g in range(NSLOT // L):
            uvec = uidx_v[pl.ds(b * NSLOT + g * L, L)]
            ivec = iidx_v[pl.ds(b * NSLOT + g * L, L)]
            lane_u = (g * L + lanes) * FW + (uvec & (FW - 1))
            lane_i = (g * L + lanes) * FW + (ivec & (FW - 1))
            acc = jnp.zeros((L,), jnp.float32)
            for tr in range(TR):
                trv = jnp.full((L,), tr, jnp.int32)
                for wr in range(WR):
                    wrv = jnp.full((L,), wr, jnp.int32)
                    u = plsc.load_gather(u_scr, [trv, wrv, lane_u])
                    v = plsc.load_gather(i_scr, [trv, wrv, lane_i])
                    acc = acc + u * v
            out_v[pl.ds(b * NSLOT + g * L, L)] = acc
        return carry

    lax.fori_loop(0, NB, batch_body, 0)

    pltpu.sync_copy(out_v, out_hbm.at[pl.ds(wid * BPW, BPW)])


def kernel(data, user_factors, item_factors):
    users = data[:, 0].astype(jnp.int32).reshape(NW, BPW)
    items = data[:, 1].astype(jnp.int32).reshape(NW, BPW)
    uq = user_factors.T.reshape(TR, WR, N_ROWS)
    iq = item_factors.T.reshape(TR, WR, N_ROWS)
    return _sc_dot(users, items, uq, iq)
